# Initial kernel scaffold; baseline (speedup 1.0000x reference)
import jax
import jax.numpy as jnp
from jax import lax
from jax.experimental import pallas as pl
from jax.experimental.pallas import tpu as pltpu
from jax.experimental.pallas import tpu_sc as plsc

N_NODES = 10000
D = 128
E_SIGN = 320000
B_NODES = 8192

N_TILES = 16
N_PAD = 10112
ROWS_PER_TILE = N_PAD // N_TILES
E_PAD = 327680
E_TILE = E_PAD // N_TILES
CHUNK = 640
SUB = CHUNK // 128
ITERS = E_TILE // CHUNK
EROWS_TILE = E_TILE // 128
NODE_ROWS_TILE = B_NODES // N_TILES // 128
SELF_ROWS_TILE = B_NODES // (2 * N_TILES) // 128


def _sc_body(x_hbm, psrc, pdst, nsrc, ndst, nodes2, z2d, z1d, ones_hbm,
             self_out, pos_out, pcnt_out, neg_out, ncnt_out,
             big_v, idx_s, idx_d, nodes_v, cnt_v, ones_v,
             ssum, scnt, sem_g, sem_s):
    c = lax.axis_index("c")
    s = lax.axis_index("s")

    pltpu.sync_copy(z2d, ssum.at[pl.ds(s * ROWS_PER_TILE, ROWS_PER_TILE)])
    pltpu.sync_copy(z1d, scnt.at[pl.ds(s * ROWS_PER_TILE, ROWS_PER_TILE)])
    pltpu.sync_copy(ones_hbm, ones_v)
    plsc.subcore_barrier()

    def accumulate(src2, dst2):
        def chunk(i, carry):
            row0 = s * EROWS_TILE + i * SUB
            pltpu.sync_copy(src2.at[pl.ds(row0, SUB)], idx_s)
            pltpu.sync_copy(dst2.at[pl.ds(row0, SUB)], idx_d)
            gathers = [
                pltpu.async_copy(x_hbm.at[idx_s.at[j]],
                                 big_v.at[pl.ds(j * 128, 128)], sem_g)
                for j in range(SUB)
            ]
            for cp in gathers:
                cp.wait()
            scatters = [
                pltpu.async_copy(big_v.at[pl.ds(j * 128, 128)],
                                 ssum.at[idx_d.at[j]], sem_s, add=True)
                for j in range(SUB)
            ]
            counts = [
                pltpu.async_copy(ones_v, scnt.at[idx_d.at[j]], sem_s, add=True)
                for j in range(SUB)
            ]
            for cp in scatters + counts:
                cp.wait()
            return carry
        lax.fori_loop(0, ITERS, chunk, 0)

    @pl.when(c == 0)
    def _():
        accumulate(psrc, pdst)

    @pl.when(c == 1)
    def _():
        accumulate(nsrc, ndst)

    plsc.subcore_barrier()

    pltpu.sync_copy(nodes2.at[pl.ds(s * NODE_ROWS_TILE, NODE_ROWS_TILE)], nodes_v)
    feat = [
        pltpu.async_copy(ssum.at[nodes_v.at[j]],
                         big_v.at[pl.ds(j * 128, 128)], sem_g)
        for j in range(NODE_ROWS_TILE)
    ]
    cnts = [
        pltpu.async_copy(scnt.at[nodes_v.at[j]],
                         cnt_v.at[pl.ds(j * 128, 128)], sem_s)
        for j in range(NODE_ROWS_TILE)
    ]
    for cp in feat + cnts:
        cp.wait()

    nchunk = NODE_ROWS_TILE * 128

    @pl.when(c == 0)
    def _():
        pltpu.sync_copy(big_v.at[pl.ds(0, nchunk)],
                        pos_out.at[pl.ds(s * nchunk, nchunk)])
        pltpu.sync_copy(cnt_v, pcnt_out.at[pl.ds(s * nchunk, nchunk)])

    @pl.when(c == 1)
    def _():
        pltpu.sync_copy(big_v.at[pl.ds(0, nchunk)],
                        neg_out.at[pl.ds(s * nchunk, nchunk)])
        pltpu.sync_copy(cnt_v, ncnt_out.at[pl.ds(s * nchunk, nchunk)])

    wid = c * N_TILES + s
    pltpu.sync_copy(nodes2.at[pl.ds(wid * SELF_ROWS_TILE, SELF_ROWS_TILE)],
                    nodes_v.at[pl.ds(0, SELF_ROWS_TILE)])
    selfs = [
        pltpu.async_copy(x_hbm.at[nodes_v.at[j]],
                         big_v.at[pl.ds(j * 128, 128)], sem_g)
        for j in range(SELF_ROWS_TILE)
    ]
    for cp in selfs:
        cp.wait()
    schunk = SELF_ROWS_TILE * 128
    pltpu.sync_copy(big_v.at[pl.ds(0, schunk)],
                    self_out.at[pl.ds(wid * schunk, schunk)])


_sc_aggregate = pl.kernel(
    _sc_body,
    out_type=(
        jax.ShapeDtypeStruct((B_NODES, D), jnp.float32),
        jax.ShapeDtypeStruct((B_NODES, D), jnp.float32),
        jax.ShapeDtypeStruct((B_NODES,), jnp.float32),
        jax.ShapeDtypeStruct((B_NODES, D), jnp.float32),
        jax.ShapeDtypeStruct((B_NODES,), jnp.float32),
    ),
    mesh=plsc.VectorSubcoreMesh(core_axis_name="c", subcore_axis_name="s"),
    scratch_types=(
        pltpu.VMEM((CHUNK, D), jnp.float32),
        pltpu.VMEM((SUB, 128), jnp.int32),
        pltpu.VMEM((SUB, 128), jnp.int32),
        pltpu.VMEM((NODE_ROWS_TILE, 128), jnp.int32),
        pltpu.VMEM((NODE_ROWS_TILE * 128,), jnp.float32),
        pltpu.VMEM((128,), jnp.float32),
        pltpu.VMEM_SHARED((N_PAD, D), jnp.float32),
        pltpu.VMEM_SHARED((N_PAD,), jnp.float32),
        pltpu.SemaphoreType.DMA,
        pltpu.SemaphoreType.DMA,
    ),
)


def _tc_body(self_ref, pos_ref, pcnt_ref, neg_ref, ncnt_ref, w_ref, o_ref):
    pos_mean = pos_ref[...] / jnp.maximum(pcnt_ref[...], 1.0)
    neg_mean = neg_ref[...] / jnp.maximum(ncnt_ref[...], 1.0)
    h = jnp.dot(self_ref[...], w_ref[0:D, :], preferred_element_type=jnp.float32)
    h = h + jnp.dot(pos_mean, w_ref[D:2 * D, :], preferred_element_type=jnp.float32)
    h = h + jnp.dot(neg_mean, w_ref[2 * D:3 * D, :], preferred_element_type=jnp.float32)
    o_ref[...] = jnp.tanh(h)


_TC_BLOCK = 512
_tc_combine = pl.pallas_call(
    _tc_body,
    grid=(B_NODES // _TC_BLOCK,),
    in_specs=[
        pl.BlockSpec((_TC_BLOCK, D), lambda i: (i, 0)),
        pl.BlockSpec((_TC_BLOCK, D), lambda i: (i, 0)),
        pl.BlockSpec((_TC_BLOCK, 1), lambda i: (i, 0)),
        pl.BlockSpec((_TC_BLOCK, D), lambda i: (i, 0)),
        pl.BlockSpec((_TC_BLOCK, 1), lambda i: (i, 0)),
        pl.BlockSpec((3 * D, D), lambda i: (0, 0)),
    ],
    out_specs=pl.BlockSpec((_TC_BLOCK, D), lambda i: (i, 0)),
    out_shape=jax.ShapeDtypeStruct((B_NODES, D), jnp.float32),
)


def kernel(x, pos_edge_index, neg_edge_index, nodes, W):
    e = pos_edge_index.shape[1]
    pad = E_PAD - e
    pad_src = jnp.zeros((pad,), jnp.int32)
    pad_dst = jnp.full((pad,), N_NODES, jnp.int32)

    def prep(edge_index):
        src = jnp.concatenate([edge_index[0], pad_src]).reshape(E_PAD // 128, 128)
        dst = jnp.concatenate([edge_index[1], pad_dst]).reshape(E_PAD // 128, 128)
        return src, dst

    psrc, pdst = prep(pos_edge_index)
    nsrc, ndst = prep(neg_edge_index)
    nodes2 = nodes.reshape(B_NODES // 128, 128)
    z2d = jnp.zeros((ROWS_PER_TILE, D), jnp.float32)
    z1d = jnp.zeros((ROWS_PER_TILE,), jnp.float32)
    ones = jnp.ones((128,), jnp.float32)

    self_f, pos_s, pos_c, neg_s, neg_c = _sc_aggregate(
        x, psrc, pdst, nsrc, ndst, nodes2, z2d, z1d, ones)
    return _tc_combine(self_f, pos_s, pos_c.reshape(B_NODES, 1),
                       neg_s, neg_c.reshape(B_NODES, 1), W)


# speedup vs baseline: 1.3353x; 1.3353x over previous
import jax
import jax.numpy as jnp
from jax import lax
from jax.experimental import pallas as pl
from jax.experimental.pallas import tpu as pltpu
from jax.experimental.pallas import tpu_sc as plsc

N_NODES = 10000
D = 128
E_SIGN = 320000
B_NODES = 8192

N_TILES = 16
HALF = 5120
ACC_ROWS = 5248
ACC_TILE = ACC_ROWS // N_TILES
CNT_ROWS = 6144
CNT_TILE = CNT_ROWS // N_TILES
DUMP = HALF
OUT_PAD = B_NODES + 128
E_PAD = 327680
E_TILE = E_PAD // N_TILES
CHUNK = 1024
SUB = CHUNK // 128
GRP = 4
ITERS = E_TILE // CHUNK
EROWS_TILE = E_TILE // 128
NODE_ROWS_TILE = B_NODES // N_TILES // 128
SELF_ROWS_TILE = 2


def _sc_body(x_hbm, psrc, pdst, nsrc, ndst, nodes_a, zacc, zcnt, ones_hbm,
             self_out, pos_out, pcnt_out, neg_out, ncnt_out,
             big_v, idx_s, idx_d, idx_dl, nodes_v, lid_blk, opos_blk,
             cntb_v, ones_v, ssum, scnt, sem_g, sem_s):
    c = lax.axis_index("c")
    s = lax.axis_index("s")

    pltpu.sync_copy(ones_hbm, ones_v)
    pltpu.sync_copy(nodes_a.at[s], nodes_v)

    selfs = [
        pltpu.async_copy(x_hbm.at[nodes_v.at[c * SELF_ROWS_TILE + j]],
                         big_v.at[pl.ds(j * 128, 128)], sem_g)
        for j in range(SELF_ROWS_TILE)
    ]
    for cp in selfs:
        cp.wait()
    nchunk = NODE_ROWS_TILE * 128
    schunk = SELF_ROWS_TILE * 128
    pltpu.sync_copy(big_v.at[pl.ds(0, schunk)],
                    self_out.at[pl.ds(s * nchunk + c * schunk, schunk)])

    def accumulate(src2, dst2, lo):
        def chunk(i, carry):
            row0 = s * EROWS_TILE + i * SUB
            pltpu.sync_copy(src2.at[pl.ds(row0, SUB)], idx_s)
            pltpu.sync_copy(dst2.at[pl.ds(row0, SUB)], idx_d)
            for r in range(SUB):
                for k in range(8):
                    v = idx_d[r, pl.ds(k * 16, 16)]
                    local = v - lo
                    ok = (local >= 0) & (local < HALF)
                    idx_dl[r, pl.ds(k * 16, 16)] = jnp.where(ok, local, DUMP)
            for g in range(SUB // GRP):
                gathers = [
                    pltpu.async_copy(x_hbm.at[idx_s.at[g * GRP + j]],
                                     big_v.at[pl.ds(j * 128, 128)], sem_g)
                    for j in range(GRP)
                ]
                for cp in gathers:
                    cp.wait()
                adds = [
                    pltpu.async_copy(big_v.at[pl.ds(j * 128, 128)],
                                     ssum.at[idx_dl.at[g * GRP + j]], sem_s,
                                     add=True)
                    for j in range(GRP)
                ] + [
                    pltpu.async_copy(ones_v, scnt.at[idx_dl.at[g * GRP + j]],
                                     sem_s, add=True)
                    for j in range(GRP)
                ]
                for cp in adds:
                    cp.wait()
            return carry
        lax.fori_loop(0, ITERS, chunk, 0)

    def emit(out_ref, cnt_ref, lo):
        lane = lax.iota(jnp.int32, 16)
        for j in range(NODE_ROWS_TILE):
            for k in range(8):
                v = nodes_v[j, pl.ds(k * 16, 16)]
                local = v - lo
                ok = (local >= 0) & (local < HALF)
                lid_blk[pl.ds(k * 16, 16)] = jnp.where(ok, local, DUMP)
                gpos = s * nchunk + j * 128 + k * 16 + lane
                opos_blk[pl.ds(k * 16, 16)] = jnp.where(ok, gpos, B_NODES)
            pltpu.async_copy(ssum.at[lid_blk],
                             big_v.at[pl.ds(0, 128)], sem_g).wait()
            pltpu.async_copy(scnt.at[lid_blk], cntb_v, sem_s).wait()
            pltpu.async_copy(big_v.at[pl.ds(0, 128)],
                             out_ref.at[opos_blk], sem_g).wait()
            pltpu.async_copy(cntb_v, cnt_ref.at[opos_blk], sem_s).wait()

    for h in range(2):
        lo = h * HALF
        pltpu.sync_copy(zacc, ssum.at[pl.ds(s * ACC_TILE, ACC_TILE)])
        pltpu.sync_copy(zcnt, scnt.at[pl.ds(s * CNT_TILE, CNT_TILE)])
        plsc.subcore_barrier()

        @pl.when(c == 0)
        def _():
            accumulate(psrc, pdst, lo)

        @pl.when(c == 1)
        def _():
            accumulate(nsrc, ndst, lo)

        plsc.subcore_barrier()

        @pl.when(c == 0)
        def _():
            emit(pos_out, pcnt_out, lo)

        @pl.when(c == 1)
        def _():
            emit(neg_out, ncnt_out, lo)

        plsc.subcore_barrier()


_sc_aggregate = pl.kernel(
    _sc_body,
    out_type=(
        jax.ShapeDtypeStruct((B_NODES, D), jnp.float32),
        jax.ShapeDtypeStruct((OUT_PAD, D), jnp.float32),
        jax.ShapeDtypeStruct((OUT_PAD,), jnp.float32),
        jax.ShapeDtypeStruct((OUT_PAD, D), jnp.float32),
        jax.ShapeDtypeStruct((OUT_PAD,), jnp.float32),
    ),
    mesh=plsc.VectorSubcoreMesh(core_axis_name="c", subcore_axis_name="s"),
    scratch_types=(
        pltpu.VMEM((GRP * 128, D), jnp.float32),
        pltpu.VMEM((SUB, 128), jnp.int32),
        pltpu.VMEM((SUB, 128), jnp.int32),
        pltpu.VMEM((SUB, 128), jnp.int32),
        pltpu.VMEM((NODE_ROWS_TILE, 128), jnp.int32),
        pltpu.VMEM((128,), jnp.int32),
        pltpu.VMEM((128,), jnp.int32),
        pltpu.VMEM((128,), jnp.float32),
        pltpu.VMEM((128,), jnp.float32),
        pltpu.VMEM_SHARED((ACC_ROWS, D), jnp.float32),
        pltpu.VMEM_SHARED((CNT_ROWS,), jnp.float32),
        pltpu.SemaphoreType.DMA,
        pltpu.SemaphoreType.DMA,
    ),
)


def _tc_body(self_ref, pos_ref, pcnt_ref, neg_ref, ncnt_ref, w_ref, o_ref):
    pos_mean = pos_ref[...] / jnp.maximum(pcnt_ref[...], 1.0)
    neg_mean = neg_ref[...] / jnp.maximum(ncnt_ref[...], 1.0)
    h = jnp.dot(self_ref[...], w_ref[0:D, :], preferred_element_type=jnp.float32)
    h = h + jnp.dot(pos_mean, w_ref[D:2 * D, :], preferred_element_type=jnp.float32)
    h = h + jnp.dot(neg_mean, w_ref[2 * D:3 * D, :], preferred_element_type=jnp.float32)
    o_ref[...] = jnp.tanh(h)


_TC_BLOCK = 512
_tc_combine = pl.pallas_call(
    _tc_body,
    grid=(B_NODES // _TC_BLOCK,),
    in_specs=[
        pl.BlockSpec((_TC_BLOCK, D), lambda i: (i, 0)),
        pl.BlockSpec((_TC_BLOCK, D), lambda i: (i, 0)),
        pl.BlockSpec((_TC_BLOCK, 1), lambda i: (i, 0)),
        pl.BlockSpec((_TC_BLOCK, D), lambda i: (i, 0)),
        pl.BlockSpec((_TC_BLOCK, 1), lambda i: (i, 0)),
        pl.BlockSpec((3 * D, D), lambda i: (0, 0)),
    ],
    out_specs=pl.BlockSpec((_TC_BLOCK, D), lambda i: (i, 0)),
    out_shape=jax.ShapeDtypeStruct((B_NODES, D), jnp.float32),
)


def kernel(x, pos_edge_index, neg_edge_index, nodes, W):
    e = pos_edge_index.shape[1]
    pad = E_PAD - e
    pad_src = jnp.zeros((pad,), jnp.int32)
    pad_dst = jnp.full((pad,), 2 * HALF, jnp.int32)

    def prep(edge_index):
        src = jnp.concatenate([edge_index[0], pad_src]).reshape(E_PAD // 128, 128)
        dst = jnp.concatenate([edge_index[1], pad_dst]).reshape(E_PAD // 128, 128)
        return src, dst

    psrc, pdst = prep(pos_edge_index)
    nsrc, ndst = prep(neg_edge_index)
    nodes_a = nodes.reshape(N_TILES, NODE_ROWS_TILE, 128)
    zacc = jnp.zeros((ACC_TILE, D), jnp.float32)
    zcnt = jnp.zeros((CNT_TILE,), jnp.float32)
    ones = jnp.ones((128,), jnp.float32)

    self_f, pos_s, pos_c, neg_s, neg_c = _sc_aggregate(
        x, psrc, pdst, nsrc, ndst, nodes_a, zacc, zcnt, ones)
    return _tc_combine(self_f, pos_s, pos_c.reshape(OUT_PAD, 1),
                       neg_s, neg_c.reshape(OUT_PAD, 1), W)
